# baseline (speedup 1.0000x reference)
import functools

import jax
import jax.numpy as jnp
from jax import lax
from jax.experimental import pallas as pl
from jax.experimental.pallas import tpu as pltpu
from jax.experimental.pallas import tpu_sc as plsc

_MARGIN = 0.1
_IGNORE_INDEX = 0

_B = 1024
_V = 100000
_NC = 2
_NS = 16
_NW = _NC * _NS
_CR = 48
_M = 8
_RPW = _CR * _M
_VS = _RPW * _NW
_CPW = _B // _NW
_NEG_INF = float("-inf")

_BH = 4000
_FB = _VS // _BH
_NBT = (_V + _BH - 1) // _BH - _FB


def _sc_body(st_hbm, gold_hbm, out_hbm,
             gold_v, buf_v, acc_v, out_v, gidx_v, gold_s, sems):
    wid = lax.axis_index("s") * _NC + lax.axis_index("c")
    row0 = wid * _RPW
    seg0 = wid * _CPW

    pltpu.sync_copy(gold_hbm, gold_v)
    for h in range(_CPW // 16):
        vec = gold_v[pl.ds(seg0 + h * 16, 16)]
        for l in range(16):
            gold_s[h * 16 + l] = vec[l]

    ninf16 = jnp.full((16,), _NEG_INF, jnp.float32)
    zero16 = jnp.zeros((16,), jnp.float32)
    for k in range(_B // 16):
        acc_v[pl.ds(k * 16, 16)] = ninf16

    def _start(t, bb):
        pltpu.async_copy(
            st_hbm.at[pl.ds(row0 + t * _CR, _CR)], buf_v.at[bb], sems.at[bb]
        )

    def _wait(bb):
        pltpu.make_async_copy(
            st_hbm.at[pl.ds(0, _CR)], buf_v.at[bb], sems.at[bb]
        ).wait()

    _start(0, 0)

    @pl.loop(0, _M, step=2)
    def _chunks(t0):
        for bb in range(2):
            t = t0 + bb
            crow0 = row0 + t * _CR

            @pl.when(t + 1 < _M)
            def _prefetch():
                _start(t + 1, 1 - bb)

            _wait(bb)

            for k4 in range(16):
                sls = [pl.ds((k4 * 4 + j) * 16, 16) for j in range(4)]
                gvs = [gold_v[sl] for sl in sls]
                init = tuple(acc_v[sl] for sl in sls)

                @pl.loop(0, _CR, init_carry=init, unroll=4)
                def _rows(s, accs, bb=bb, sls=sls, gvs=gvs, crow0=crow0):
                    rowid = crow0 + s
                    return tuple(
                        jnp.maximum(a, jnp.where(gv == rowid, _NEG_INF,
                                                 buf_v[bb, s, sl]))
                        for a, gv, sl in zip(accs, gvs, sls)
                    )

                for sl, a in zip(sls, _rows):
                    acc_v[sl] = a

    for h in range(_CPW // 16):
        gidx_v[pl.ds(h * 16, 16)] = gold_v[pl.ds(seg0 + h * 16, 16)]
    pltpu.async_copy(
        st_hbm.at[gidx_v], buf_v.at[0, pl.ds(0, _CPW)], sems.at[0]
    ).wait()

    for k in range(_B // 16):
        out_v[0, pl.ds(k * 16, 16)] = acc_v[pl.ds(k * 16, 16)]
        out_v[1, pl.ds(k * 16, 16)] = zero16
    lane = lax.iota(jnp.int32, 16)
    for h in range(_CPW // 16):
        gacc = zero16
        for l in range(16):
            j = h * 16 + l
            g_j = gold_s[j]
            gate = jnp.where(g_j < _VS, 1.0, 0.0)
            grp = buf_v[0, j, pl.ds(seg0 + h * 16, 16)]
            gacc = gacc + jnp.where(lane == l, grp, 0.0) * gate
        out_v[1, pl.ds(seg0 + h * 16, 16)] = gacc

    pltpu.sync_copy(out_v, out_hbm.at[pl.ds(wid * 8, 8)])


def _tc_main(x_ref, g_ref, neg_ref, gold_ref, neg_acc, gold_acc, *, nb):
    i = pl.program_id(0)

    @pl.when(i == 0)
    def _init():
        neg_acc[...] = jnp.full_like(neg_acc, -jnp.inf)
        gold_acc[...] = jnp.zeros_like(gold_acc)

    x = x_ref[...]
    row = (_FB + i) * _BH + jax.lax.broadcasted_iota(jnp.int32, x.shape, 0)
    g = g_ref[...]
    valid = (row >= _VS) & (row < _V)
    is_gold = row == g
    neg = jnp.max(jnp.where(is_gold | ~valid, -jnp.inf, x), axis=0,
                  keepdims=True)
    neg_acc[...] = jnp.maximum(neg_acc[...], neg)
    gold_acc[...] += jnp.sum(jnp.where(is_gold & valid, x, 0.0), axis=0,
                             keepdims=True)

    @pl.when(i == nb - 1)
    def _final():
        neg_ref[...] = neg_acc[...]
        gold_ref[...] = gold_acc[...]


def _combine(sc_ref, tcn_ref, tcg_ref, g_ref, o_ref, *, b):
    sc = sc_ref[...]
    rowmod = jax.lax.broadcasted_iota(jnp.int32, sc.shape, 0) % 8
    sc_neg = jnp.max(jnp.where(rowmod == 0, sc, -jnp.inf), axis=0,
                     keepdims=True)
    sc_gold = jnp.sum(jnp.where(rowmod == 1, sc, 0.0), axis=0, keepdims=True)
    g = g_ref[...]
    neg = jnp.maximum(sc_neg, tcn_ref[...])
    golds = sc_gold + tcg_ref[...]
    loss = jnp.maximum(_MARGIN + neg - golds, 0.0)
    loss = loss * (g != _IGNORE_INDEX).astype(loss.dtype)
    o_ref[0, 0] = jnp.sum(loss) / b


@jax.jit
def kernel(scores, gold):
    b, v = scores.shape
    st = scores.T
    gold32 = gold.astype(jnp.int32)
    gold2 = gold32.reshape(1, b)

    sc_out = pl.kernel(
        _sc_body,
        out_type=jax.ShapeDtypeStruct((8 * _NW, _B), jnp.float32),
        mesh=plsc.VectorSubcoreMesh(core_axis_name="c", subcore_axis_name="s"),
        scratch_types=[
            pltpu.VMEM((_B,), jnp.int32),
            pltpu.VMEM((2, _CR, _B), jnp.float32),
            pltpu.VMEM((_B,), jnp.float32),
            pltpu.VMEM((8, _B), jnp.float32),
            pltpu.VMEM((_CPW,), jnp.int32),
            pltpu.SMEM((_CPW,), jnp.int32),
            pltpu.SemaphoreType.DMA((2,)),
        ],
    )(st, gold32)

    tc_neg, tc_gold = pl.pallas_call(
        functools.partial(_tc_main, nb=_NBT),
        grid=(_NBT,),
        in_specs=[
            pl.BlockSpec((_BH, b), lambda i: (_FB + i, 0)),
            pl.BlockSpec((1, b), lambda i: (0, 0)),
        ],
        out_specs=[
            pl.BlockSpec((1, b), lambda i: (0, 0)),
            pl.BlockSpec((1, b), lambda i: (0, 0)),
        ],
        out_shape=[
            jax.ShapeDtypeStruct((1, b), jnp.float32),
            jax.ShapeDtypeStruct((1, b), jnp.float32),
        ],
        scratch_shapes=[
            pltpu.VMEM((1, b), jnp.float32),
            pltpu.VMEM((1, b), jnp.float32),
        ],
    )(st, gold2)

    out = pl.pallas_call(
        functools.partial(_combine, b=b),
        grid=(1,),
        in_specs=[
            pl.BlockSpec((8 * _NW, b), lambda i: (0, 0)),
            pl.BlockSpec((1, b), lambda i: (0, 0)),
            pl.BlockSpec((1, b), lambda i: (0, 0)),
            pl.BlockSpec((1, b), lambda i: (0, 0)),
        ],
        out_specs=pl.BlockSpec(memory_space=pltpu.SMEM),
        out_shape=jax.ShapeDtypeStruct((1, 1), jnp.float32),
    )(sc_out, tc_neg, tc_gold, gold2)
    return out[0, 0]


# speedup vs baseline: 1.3709x; 1.3709x over previous
import functools

import jax
import jax.numpy as jnp
from jax import lax
from jax.experimental import pallas as pl
from jax.experimental.pallas import tpu as pltpu
from jax.experimental.pallas import tpu_sc as plsc

_MARGIN = 0.1
_IGNORE_INDEX = 0

_B = 1024
_V = 100000
_NC = 2
_NS = 16
_NW = _NC * _NS
_CR = 40
_M = 25
_RPW = _CR * _M
_VS = _RPW * _NW
_CPW = _B // _NW
_NEG_INF = float("-inf")

_BH = 4000
_FB = _VS // _BH
_NBT = _V // _BH - _FB


def _sc_body(st_hbm, gold_hbm, out_hbm,
             gold_v, buf_v, acc_v, out_v, gidx_v, gold_s, sems):
    wid = lax.axis_index("s") * _NC + lax.axis_index("c")
    row0 = wid * _RPW
    seg0 = wid * _CPW

    pltpu.sync_copy(gold_hbm, gold_v)
    for h in range(_CPW // 16):
        vec = gold_v[pl.ds(seg0 + h * 16, 16)]
        for l in range(16):
            gold_s[h * 16 + l] = vec[l]

    ninf16 = jnp.full((16,), _NEG_INF, jnp.float32)
    zero16 = jnp.zeros((16,), jnp.float32)
    for k in range(_B // 16):
        acc_v[pl.ds(k * 16, 16)] = ninf16

    def _start(t, bb):
        pltpu.async_copy(
            st_hbm.at[pl.ds(row0 + t * _CR, _CR)], buf_v.at[bb], sems.at[bb]
        )

    def _wait(bb):
        pltpu.make_async_copy(
            st_hbm.at[pl.ds(0, _CR)], buf_v.at[bb], sems.at[bb]
        ).wait()

    def _consume(t, bb, prefetch):
        crow0 = row0 + t * _CR

        if prefetch:
            _start(t + 1, 1 - bb)

        _wait(bb)

        for k4 in range(16):
            sls = [pl.ds((k4 * 4 + j) * 16, 16) for j in range(4)]
            gvs = [gold_v[sl] for sl in sls]
            init = tuple(acc_v[sl] for sl in sls)

            @pl.loop(0, _CR, init_carry=init, unroll=4)
            def _rows(s, accs, bb=bb, sls=sls, gvs=gvs, crow0=crow0):
                rowid = crow0 + s
                return tuple(
                    jnp.maximum(a, jnp.where(gv == rowid, _NEG_INF,
                                             buf_v[bb, s, sl]))
                    for a, gv, sl in zip(accs, gvs, sls)
                )

            for sl, a in zip(sls, _rows):
                acc_v[sl] = a

    _start(0, 0)

    @pl.loop(0, _M - 1, step=2)
    def _chunks(t0):
        for bb in range(2):
            _consume(t0 + bb, bb, True)

    _consume(_M - 1, (_M - 1) % 2, False)

    for h in range(_CPW // 16):
        gidx_v[pl.ds(h * 16, 16)] = gold_v[pl.ds(seg0 + h * 16, 16)]
    pltpu.async_copy(
        st_hbm.at[gidx_v], buf_v.at[0, pl.ds(0, _CPW)], sems.at[0]
    ).wait()

    for k in range(_B // 16):
        out_v[0, pl.ds(k * 16, 16)] = acc_v[pl.ds(k * 16, 16)]
        out_v[1, pl.ds(k * 16, 16)] = zero16
    lane = lax.iota(jnp.int32, 16)
    for h in range(_CPW // 16):
        gacc = zero16
        for l in range(16):
            j = h * 16 + l
            g_j = gold_s[j]
            gate = jnp.where(g_j < _VS, 1.0, 0.0)
            grp = buf_v[0, j, pl.ds(seg0 + h * 16, 16)]
            gacc = gacc + jnp.where(lane == l, grp, 0.0) * gate
        out_v[1, pl.ds(seg0 + h * 16, 16)] = gacc

    pltpu.sync_copy(out_v, out_hbm.at[pl.ds(wid * 8, 8)])


def _tc_main(x_ref, g_ref, neg_ref, gold_ref, neg_acc, gold_acc, *, nb):
    i = pl.program_id(0)

    @pl.when(i == 0)
    def _init():
        neg_acc[...] = jnp.full_like(neg_acc, -jnp.inf)
        gold_acc[...] = jnp.zeros_like(gold_acc)

    x = x_ref[...]
    row = (_FB + i) * _BH + jax.lax.broadcasted_iota(jnp.int32, x.shape, 0)
    g = g_ref[...]
    is_gold = row == g
    neg = jnp.max(jnp.where(is_gold, -jnp.inf, x), axis=0, keepdims=True)
    neg_acc[...] = jnp.maximum(neg_acc[...], neg)
    gold_acc[...] += jnp.sum(jnp.where(is_gold, x, 0.0), axis=0, keepdims=True)

    @pl.when(i == nb - 1)
    def _final():
        neg_ref[...] = neg_acc[...]
        gold_ref[...] = gold_acc[...]


def _combine(sc_ref, tcn_ref, tcg_ref, g_ref, o_ref, *, b):
    sc = sc_ref[...]
    rowmod = jax.lax.broadcasted_iota(jnp.int32, sc.shape, 0) % 8
    sc_neg = jnp.max(jnp.where(rowmod == 0, sc, -jnp.inf), axis=0,
                     keepdims=True)
    sc_gold = jnp.sum(jnp.where(rowmod == 1, sc, 0.0), axis=0, keepdims=True)
    g = g_ref[...]
    neg = jnp.maximum(sc_neg, tcn_ref[...])
    golds = sc_gold + tcg_ref[...]
    loss = jnp.maximum(_MARGIN + neg - golds, 0.0)
    loss = loss * (g != _IGNORE_INDEX).astype(loss.dtype)
    o_ref[0, 0] = jnp.sum(loss) / b


@jax.jit
def kernel(scores, gold):
    b, v = scores.shape
    st = scores.T
    gold32 = gold.astype(jnp.int32)
    gold2 = gold32.reshape(1, b)

    sc_out = pl.kernel(
        _sc_body,
        out_type=jax.ShapeDtypeStruct((8 * _NW, _B), jnp.float32),
        mesh=plsc.VectorSubcoreMesh(core_axis_name="c", subcore_axis_name="s"),
        scratch_types=[
            pltpu.VMEM((_B,), jnp.int32),
            pltpu.VMEM((2, _CR, _B), jnp.float32),
            pltpu.VMEM((_B,), jnp.float32),
            pltpu.VMEM((8, _B), jnp.float32),
            pltpu.VMEM((_CPW,), jnp.int32),
            pltpu.SMEM((_CPW,), jnp.int32),
            pltpu.SemaphoreType.DMA((2,)),
        ],
    )(st, gold32)

    tc_neg, tc_gold = pl.pallas_call(
        functools.partial(_tc_main, nb=_NBT),
        grid=(_NBT,),
        in_specs=[
            pl.BlockSpec((_BH, b), lambda i: (_FB + i, 0)),
            pl.BlockSpec((1, b), lambda i: (0, 0)),
        ],
        out_specs=[
            pl.BlockSpec((1, b), lambda i: (0, 0)),
            pl.BlockSpec((1, b), lambda i: (0, 0)),
        ],
        out_shape=[
            jax.ShapeDtypeStruct((1, b), jnp.float32),
            jax.ShapeDtypeStruct((1, b), jnp.float32),
        ],
        scratch_shapes=[
            pltpu.VMEM((1, b), jnp.float32),
            pltpu.VMEM((1, b), jnp.float32),
        ],
    )(st, gold2)

    out = pl.pallas_call(
        functools.partial(_combine, b=b),
        grid=(1,),
        in_specs=[
            pl.BlockSpec((8 * _NW, b), lambda i: (0, 0)),
            pl.BlockSpec((1, b), lambda i: (0, 0)),
            pl.BlockSpec((1, b), lambda i: (0, 0)),
            pl.BlockSpec((1, b), lambda i: (0, 0)),
        ],
        out_specs=pl.BlockSpec(memory_space=pltpu.SMEM),
        out_shape=jax.ShapeDtypeStruct((1, 1), jnp.float32),
    )(sc_out, tc_neg, tc_gold, gold2)
    return out[0, 0]
